# Initial kernel scaffold; baseline (speedup 1.0000x reference)
import jax
import jax.numpy as jnp
from jax.experimental import pallas as pl


def kernel(x, edge_index, pos_edge_index, neg_edge_index, params):
    raise NotImplementedError("write your pallas kernel here")


# speedup vs baseline: 4.4128x; 4.4128x over previous
import functools

import jax
import jax.numpy as jnp
from jax import lax
from jax.experimental import pallas as pl
from jax.experimental.pallas import tpu as pltpu
from jax.experimental.pallas import tpu_sc as plsc

_NC = 2
_NS = 16
_NW = _NC * _NS


def _chunk_size(per_worker, cap=128):
    for c in range(cap, 7, -8):
        if per_worker % c == 0:
            return c
    raise ValueError(f"no aligned chunk size for {per_worker}")


@functools.lru_cache(maxsize=None)
def _make_edge_agg(N, D, E, with_deg):
    lanes = 16
    rpt_unit = _NS * 8
    N_pad = ((N + rpt_unit - 1) // rpt_unit) * rpt_unit
    RPT = N_pad // _NS
    EPW = E // _NW
    assert E % _NW == 0
    CH = _chunk_size(EPW)
    NCH = EPW // CH
    ZR = min(128, RPT)
    RPT16 = ((RPT + 15) // 16) * 16

    out_type = [jax.ShapeDtypeStruct((_NC, N_pad, D), jnp.float32)]
    scratch = [
        pltpu.VMEM((CH,), jnp.int32),
        pltpu.VMEM((CH,), jnp.int32),
        pltpu.VMEM((CH, D), jnp.float32),
        pltpu.VMEM((ZR, D), jnp.float32),
        pltpu.VMEM_SHARED((N_pad, D), jnp.float32),
        pltpu.SemaphoreType.DMA,
    ]
    if with_deg:
        out_type.append(jax.ShapeDtypeStruct((_NC * N_pad,), jnp.float32))
        scratch += [
            pltpu.VMEM((CH,), jnp.float32),
            pltpu.VMEM((RPT16,), jnp.float32),
            pltpu.VMEM_SHARED((N_pad,), jnp.float32),
        ]

    mesh = plsc.VectorSubcoreMesh(core_axis_name="c", subcore_axis_name="s")

    def body(h_hbm, src_hbm, dst_hbm, *refs):
        if with_deg:
            (agg_out, deg_out, src_v, dst_v, rows_v, zbuf, agg_sh, sem,
             ones_v, zdeg, deg_sh) = refs
        else:
            agg_out, src_v, dst_v, rows_v, zbuf, agg_sh, sem = refs
        cid = lax.axis_index("c")
        sid = lax.axis_index("s")
        wid = cid * _NS + sid
        row0 = sid * RPT

        zero16 = jnp.zeros((16,), jnp.float32)

        def zrow(r, _):
            for j in range(D // lanes):
                zbuf[r, pl.ds(j * lanes, lanes)] = zero16
            return 0
        lax.fori_loop(0, ZR, zrow, 0)

        off = 0
        while off < RPT:
            sz = min(ZR, RPT - off)
            pltpu.sync_copy(zbuf.at[pl.ds(0, sz), :],
                            agg_sh.at[pl.ds(row0 + off, sz), :])
            off += sz

        if with_deg:
            one16 = jnp.ones((16,), jnp.float32)
            for j in range(CH // 16):
                ones_v[pl.ds(j * 16, 16)] = one16

            def zdrow(r, _):
                zdeg[pl.ds(r * 16, 16)] = zero16
                return 0
            lax.fori_loop(0, RPT16 // 16, zdrow, 0)
            pltpu.sync_copy(zdeg.at[pl.ds(0, RPT)],
                            deg_sh.at[pl.ds(row0, RPT)])

        plsc.subcore_barrier()

        ebase = wid * EPW

        def echunk(i, _):
            b = ebase + i * CH
            pltpu.sync_copy(src_hbm.at[pl.ds(b, CH)], src_v)
            pltpu.sync_copy(dst_hbm.at[pl.ds(b, CH)], dst_v)
            pltpu.async_copy(h_hbm.at[src_v], rows_v, sem).wait()
            pltpu.sync_copy(rows_v, agg_sh.at[dst_v], add=True)
            if with_deg:
                pltpu.sync_copy(ones_v, deg_sh.at[dst_v], add=True)
            return 0
        lax.fori_loop(0, NCH, echunk, 0)

        plsc.subcore_barrier()

        off = 0
        while off < RPT:
            sz = min(ZR, RPT - off)
            pltpu.sync_copy(agg_sh.at[pl.ds(row0 + off, sz), :],
                            zbuf.at[pl.ds(0, sz), :])
            pltpu.sync_copy(zbuf.at[pl.ds(0, sz), :],
                            agg_out.at[cid, pl.ds(row0 + off, sz), :])
            off += sz
        if with_deg:
            pltpu.sync_copy(deg_sh.at[pl.ds(row0, RPT)], zdeg.at[pl.ds(0, RPT)])
            pltpu.sync_copy(zdeg.at[pl.ds(0, RPT)],
                            deg_out.at[pl.ds(cid * N_pad + row0, RPT)])

    return pl.kernel(body, out_type=tuple(out_type), mesh=mesh,
                     scratch_types=tuple(scratch))


@functools.lru_cache(maxsize=None)
def _make_pair_gather(N, D, PG):
    PPW = PG // _NW
    assert PG % _NW == 0
    CH = _chunk_size(PPW)
    NCH = PPW // CH
    mesh = plsc.VectorSubcoreMesh(core_axis_name="c", subcore_axis_name="s")
    out_type = (jax.ShapeDtypeStruct((PG, D), jnp.float32),
                jax.ShapeDtypeStruct((PG, D), jnp.float32))
    scratch = (
        pltpu.VMEM((CH,), jnp.int32),
        pltpu.VMEM((CH,), jnp.int32),
        pltpu.VMEM((CH, D), jnp.float32),
        pltpu.SemaphoreType.DMA,
    )

    def body(h_hbm, sidx_hbm, didx_hbm, hs_out, hd_out, si_v, di_v, rows_v, sem):
        cid = lax.axis_index("c")
        sid = lax.axis_index("s")
        wid = cid * _NS + sid
        base = wid * PPW

        def chunk(i, _):
            b = base + i * CH
            pltpu.sync_copy(sidx_hbm.at[pl.ds(b, CH)], si_v)
            pltpu.async_copy(h_hbm.at[si_v], rows_v, sem).wait()
            pltpu.sync_copy(rows_v, hs_out.at[pl.ds(b, CH), :])
            pltpu.sync_copy(didx_hbm.at[pl.ds(b, CH)], di_v)
            pltpu.async_copy(h_hbm.at[di_v], rows_v, sem).wait()
            pltpu.sync_copy(rows_v, hd_out.at[pl.ds(b, CH), :])
            return 0
        lax.fori_loop(0, NCH, chunk, 0)

    return pl.kernel(body, out_type=out_type, mesh=mesh, scratch_types=scratch)


def _bn_relu(z, g, b):
    m = jnp.mean(z, axis=0, keepdims=True)
    v = jnp.mean((z - m) ** 2, axis=0, keepdims=True)
    return jnp.maximum((z - m) / jnp.sqrt(v + 1e-5) * g + b, 0.0)


def _mlp_layer(h, aggp, degp, W1, g1, b1, W2, g2, b2, outer):
    N, D = h.shape
    H = W1.shape[1]

    def body(h_ref, agg_ref, deg_ref, W1_ref, g1_ref, b1_ref, W2_ref,
             g2_ref, b2_ref, *rest):
        if outer is not None:
            og_ref, ob_ref, out_ref = rest
        else:
            (out_ref,) = rest
        agg = agg_ref[0, :N, :] + agg_ref[1, :N, :]
        deg = deg_ref[0, :N, :] + deg_ref[1, :N, :]
        r = h_ref[...] + agg / jnp.maximum(deg, 1.0)
        z = jnp.dot(r, W1_ref[...], preferred_element_type=jnp.float32)
        z = _bn_relu(z, g1_ref[...], b1_ref[...])
        z = jnp.dot(z, W2_ref[...], preferred_element_type=jnp.float32)
        z = _bn_relu(z, g2_ref[...], b2_ref[...])
        if outer is not None:
            z = _bn_relu(z, og_ref[...], ob_ref[...])
        out_ref[...] = z

    args = [h, aggp, degp, W1, g1.reshape(1, -1), b1.reshape(1, -1),
            W2, g2.reshape(1, -1), b2.reshape(1, -1)]
    if outer is not None:
        og, ob = outer
        args += [og.reshape(1, -1), ob.reshape(1, -1)]
    return pl.pallas_call(
        body, out_shape=jax.ShapeDtypeStruct((N, H), jnp.float32))(*args)


def _predictor(hs, hd, M, P, W1, b1, W2, b2, W3, b3):
    def body(hs_ref, hd_ref, w1, b1r, w2, b2r, w3, b3r, out_ref):
        t = hs_ref[:M, :] * hd_ref[:M, :]
        t = jnp.maximum(
            jnp.dot(t, w1[...], preferred_element_type=jnp.float32) + b1r[...],
            0.0)
        t = jnp.maximum(
            jnp.dot(t, w2[...], preferred_element_type=jnp.float32) + b2r[...],
            0.0)
        t = jnp.dot(t, w3[...], preferred_element_type=jnp.float32) + b3r[...]
        out_ref[0] = t[:P]
        out_ref[1] = t[P:]

    return pl.pallas_call(
        body, out_shape=jax.ShapeDtypeStruct((2, P, 1), jnp.float32))(
            hs, hd, W1, b1.reshape(1, -1), W2, b2.reshape(1, -1),
            W3, b3.reshape(1, -1))


def kernel(x, edge_index, pos_edge_index, neg_edge_index, params):
    N, D = x.shape
    E = edge_index.shape[1]
    P = pos_edge_index.shape[1]
    layers = params['layers']
    outer_bn = params['outer_bn']
    pp = params['pred']
    L = len(layers)

    src = edge_index[0]
    dst = edge_index[1]

    agg_deg = _make_edge_agg(N, D, E, True)
    agg_only = _make_edge_agg(N, D, E, False)

    h = x
    degp = None
    for l in range(L):
        p = layers[l]
        if l == 0:
            aggp, degp = agg_deg(h, src, dst)
            degp = degp.reshape(2, -1)[:, :, None]
        else:
            (aggp,) = agg_only(h, src, dst)
        outer = ((outer_bn[l]['g'], outer_bn[l]['b'])
                 if l != L - 1 else None)
        h = _mlp_layer(h, aggp, degp, p['W1'], p['bn1_g'], p['bn1_b'],
                       p['W2'], p['bn2_g'], p['bn2_b'], outer)

    M = 2 * P
    PG = ((M + _NW * 128 - 1) // (_NW * 128)) * (_NW * 128)
    pad = jnp.zeros((PG - M,), jnp.int32)
    sidx = jnp.concatenate([pos_edge_index[0], neg_edge_index[0], pad])
    didx = jnp.concatenate([pos_edge_index[1], neg_edge_index[1], pad])
    hs, hd = _make_pair_gather(N, D, PG)(h, sidx, didx)

    return _predictor(hs, hd, M, P, pp['W1'], pp['b1'], pp['W2'], pp['b2'],
                      pp['W3'], pp['b3'])


# speedup vs baseline: 6.6537x; 1.5078x over previous
import functools

import jax
import jax.numpy as jnp
from jax import lax
from jax.experimental import pallas as pl
from jax.experimental.pallas import tpu as pltpu
from jax.experimental.pallas import tpu_sc as plsc

_NC = 2
_NS = 16
_NW = _NC * _NS


def _chunk_size(per_worker, cap=128):
    for c in range(cap, 7, -8):
        if per_worker % c == 0:
            return c
    raise ValueError(f"no aligned chunk size for {per_worker}")


@functools.lru_cache(maxsize=None)
def _make_edge_agg(N, D, E, with_deg):
    lanes = 16
    rpt_unit = _NS * 8
    N_pad = ((N + rpt_unit - 1) // rpt_unit) * rpt_unit
    RPT = N_pad // _NS
    EPW = E // _NW
    assert E % _NW == 0
    CH = _chunk_size(EPW)
    NCH = EPW // CH
    ZR = min(128, RPT)
    RPT16 = ((RPT + 15) // 16) * 16

    out_type = [jax.ShapeDtypeStruct((_NC, N_pad, D), jnp.float32)]
    scratch = [
        pltpu.VMEM((CH,), jnp.int32),
        pltpu.VMEM((CH,), jnp.int32),
        pltpu.VMEM((CH, D), jnp.float32),
        pltpu.SemaphoreType.DMA,
        pltpu.VMEM((CH,), jnp.int32),
        pltpu.VMEM((CH,), jnp.int32),
        pltpu.VMEM((CH, D), jnp.float32),
        pltpu.SemaphoreType.DMA,
        pltpu.VMEM((ZR, D), jnp.float32),
        pltpu.VMEM_SHARED((N_pad, D), jnp.float32),
    ]
    if with_deg:
        out_type.append(jax.ShapeDtypeStruct((_NC * N_pad,), jnp.float32))
        scratch += [
            pltpu.VMEM((CH,), jnp.float32),
            pltpu.VMEM((RPT16,), jnp.float32),
            pltpu.VMEM_SHARED((N_pad,), jnp.float32),
        ]

    mesh = plsc.VectorSubcoreMesh(core_axis_name="c", subcore_axis_name="s")

    def body(h_hbm, src_hbm, dst_hbm, *refs):
        if with_deg:
            (agg_out, deg_out, s0, d0, r0, sem0, s1, d1, r1, sem1, zbuf,
             agg_sh, ones_v, zdeg, deg_sh) = refs
        else:
            (agg_out, s0, d0, r0, sem0, s1, d1, r1, sem1, zbuf,
             agg_sh) = refs
        cid = lax.axis_index("c")
        sid = lax.axis_index("s")
        wid = cid * _NS + sid
        row0 = sid * RPT

        zero16 = jnp.zeros((16,), jnp.float32)

        def zrow(r, _):
            for j in range(D // lanes):
                zbuf[r, pl.ds(j * lanes, lanes)] = zero16
            return 0
        lax.fori_loop(0, ZR, zrow, 0)

        off = 0
        while off < RPT:
            sz = min(ZR, RPT - off)
            pltpu.sync_copy(zbuf.at[pl.ds(0, sz), :],
                            agg_sh.at[pl.ds(row0 + off, sz), :])
            off += sz

        if with_deg:
            one16 = jnp.ones((16,), jnp.float32)
            for j in range(CH // 16):
                ones_v[pl.ds(j * 16, 16)] = one16

            def zdrow(r, _):
                zdeg[pl.ds(r * 16, 16)] = zero16
                return 0
            lax.fori_loop(0, RPT16 // 16, zdrow, 0)
            pltpu.sync_copy(zdeg.at[pl.ds(0, RPT)],
                            deg_sh.at[pl.ds(row0, RPT)])

        plsc.subcore_barrier()

        ebase = wid * EPW
        bufs = ((s0, d0, r0, sem0), (s1, d1, r1, sem1))

        def load_and_gather(c, buf):
            s_v, d_v, r_v, sem = buf
            b = ebase + c * CH
            pltpu.sync_copy(src_hbm.at[pl.ds(b, CH)], s_v)
            pltpu.sync_copy(dst_hbm.at[pl.ds(b, CH)], d_v)
            pltpu.async_copy(h_hbm.at[s_v], r_v, sem)

        def finish(buf):
            s_v, d_v, r_v, sem = buf
            pltpu.make_async_copy(h_hbm.at[s_v], r_v, sem).wait()
            pltpu.sync_copy(r_v, agg_sh.at[d_v], add=True)
            if with_deg:
                pltpu.sync_copy(ones_v, deg_sh.at[d_v], add=True)

        if NCH % 2 == 1:
            load_and_gather(0, bufs[0])

            def pipe(jj, _):
                c = 2 * jj
                load_and_gather(c + 1, bufs[1])
                finish(bufs[0])
                load_and_gather(c + 2, bufs[0])
                finish(bufs[1])
                return 0
            lax.fori_loop(0, (NCH - 1) // 2, pipe, 0)
            finish(bufs[0])
        else:
            def echunk(i, _):
                load_and_gather(i, bufs[0])
                finish(bufs[0])
                return 0
            lax.fori_loop(0, NCH, echunk, 0)

        plsc.subcore_barrier()

        off = 0
        while off < RPT:
            sz = min(ZR, RPT - off)
            pltpu.sync_copy(agg_sh.at[pl.ds(row0 + off, sz), :],
                            zbuf.at[pl.ds(0, sz), :])
            pltpu.sync_copy(zbuf.at[pl.ds(0, sz), :],
                            agg_out.at[cid, pl.ds(row0 + off, sz), :])
            off += sz
        if with_deg:
            pltpu.sync_copy(deg_sh.at[pl.ds(row0, RPT)], zdeg.at[pl.ds(0, RPT)])
            pltpu.sync_copy(zdeg.at[pl.ds(0, RPT)],
                            deg_out.at[pl.ds(cid * N_pad + row0, RPT)])

    return pl.kernel(body, out_type=tuple(out_type), mesh=mesh,
                     scratch_types=tuple(scratch))


@functools.lru_cache(maxsize=None)
def _make_pair_gather(N, D, PG):
    PPW = PG // _NW
    assert PG % _NW == 0
    CH = _chunk_size(PPW)
    NCH = PPW // CH
    mesh = plsc.VectorSubcoreMesh(core_axis_name="c", subcore_axis_name="s")
    out_type = (jax.ShapeDtypeStruct((PG, D), jnp.float32),
                jax.ShapeDtypeStruct((PG, D), jnp.float32))
    scratch = (
        pltpu.VMEM((CH,), jnp.int32),
        pltpu.VMEM((CH,), jnp.int32),
        pltpu.VMEM((CH, D), jnp.float32),
        pltpu.SemaphoreType.DMA,
    )

    def body(h_hbm, sidx_hbm, didx_hbm, hs_out, hd_out, si_v, di_v, rows_v, sem):
        cid = lax.axis_index("c")
        sid = lax.axis_index("s")
        wid = cid * _NS + sid
        base = wid * PPW

        def chunk(i, _):
            b = base + i * CH
            pltpu.sync_copy(sidx_hbm.at[pl.ds(b, CH)], si_v)
            pltpu.async_copy(h_hbm.at[si_v], rows_v, sem).wait()
            pltpu.sync_copy(rows_v, hs_out.at[pl.ds(b, CH), :])
            pltpu.sync_copy(didx_hbm.at[pl.ds(b, CH)], di_v)
            pltpu.async_copy(h_hbm.at[di_v], rows_v, sem).wait()
            pltpu.sync_copy(rows_v, hd_out.at[pl.ds(b, CH), :])
            return 0
        lax.fori_loop(0, NCH, chunk, 0)

    return pl.kernel(body, out_type=out_type, mesh=mesh, scratch_types=scratch)


def _bn_relu(z, g, b):
    m = jnp.mean(z, axis=0, keepdims=True)
    v = jnp.mean((z - m) ** 2, axis=0, keepdims=True)
    return jnp.maximum((z - m) / jnp.sqrt(v + 1e-5) * g + b, 0.0)


def _mlp_layer(h, aggp, degp, W1, g1, b1, W2, g2, b2, outer):
    N, D = h.shape
    H = W1.shape[1]

    def body(h_ref, agg_ref, deg_ref, W1_ref, g1_ref, b1_ref, W2_ref,
             g2_ref, b2_ref, *rest):
        if outer is not None:
            og_ref, ob_ref, out_ref = rest
        else:
            (out_ref,) = rest
        agg = agg_ref[0, :N, :] + agg_ref[1, :N, :]
        deg = deg_ref[0, :N, :] + deg_ref[1, :N, :]
        r = h_ref[...] + agg / jnp.maximum(deg, 1.0)
        z = jnp.dot(r, W1_ref[...], preferred_element_type=jnp.float32)
        z = _bn_relu(z, g1_ref[...], b1_ref[...])
        z = jnp.dot(z, W2_ref[...], preferred_element_type=jnp.float32)
        z = _bn_relu(z, g2_ref[...], b2_ref[...])
        if outer is not None:
            z = _bn_relu(z, og_ref[...], ob_ref[...])
        out_ref[...] = z

    args = [h, aggp, degp, W1, g1.reshape(1, -1), b1.reshape(1, -1),
            W2, g2.reshape(1, -1), b2.reshape(1, -1)]
    if outer is not None:
        og, ob = outer
        args += [og.reshape(1, -1), ob.reshape(1, -1)]
    return pl.pallas_call(
        body, out_shape=jax.ShapeDtypeStruct((N, H), jnp.float32))(*args)


def _predictor(hs, hd, M, P, W1, b1, W2, b2, W3, b3):
    def body(hs_ref, hd_ref, w1, b1r, w2, b2r, w3, b3r, out_ref):
        t = hs_ref[:M, :] * hd_ref[:M, :]
        t = jnp.maximum(
            jnp.dot(t, w1[...], preferred_element_type=jnp.float32) + b1r[...],
            0.0)
        t = jnp.maximum(
            jnp.dot(t, w2[...], preferred_element_type=jnp.float32) + b2r[...],
            0.0)
        t = jnp.dot(t, w3[...], preferred_element_type=jnp.float32) + b3r[...]
        out_ref[0] = t[:P]
        out_ref[1] = t[P:]

    return pl.pallas_call(
        body, out_shape=jax.ShapeDtypeStruct((2, P, 1), jnp.float32))(
            hs, hd, W1, b1.reshape(1, -1), W2, b2.reshape(1, -1),
            W3, b3.reshape(1, -1))


def kernel(x, edge_index, pos_edge_index, neg_edge_index, params):
    N, D = x.shape
    E = edge_index.shape[1]
    P = pos_edge_index.shape[1]
    layers = params['layers']
    outer_bn = params['outer_bn']
    pp = params['pred']
    L = len(layers)

    src = edge_index[0]
    dst = edge_index[1]

    agg_deg = _make_edge_agg(N, D, E, True)
    agg_only = _make_edge_agg(N, D, E, False)

    h = x
    degp = None
    for l in range(L):
        p = layers[l]
        if l == 0:
            aggp, degp = agg_deg(h, src, dst)
            degp = degp.reshape(2, -1)[:, :, None]
        else:
            (aggp,) = agg_only(h, src, dst)
        outer = ((outer_bn[l]['g'], outer_bn[l]['b'])
                 if l != L - 1 else None)
        h = _mlp_layer(h, aggp, degp, p['W1'], p['bn1_g'], p['bn1_b'],
                       p['W2'], p['bn2_g'], p['bn2_b'], outer)

    M = 2 * P
    PG = ((M + _NW * 128 - 1) // (_NW * 128)) * (_NW * 128)
    pad = jnp.zeros((PG - M,), jnp.int32)
    sidx = jnp.concatenate([pos_edge_index[0], neg_edge_index[0], pad])
    didx = jnp.concatenate([pos_edge_index[1], neg_edge_index[1], pad])
    hs, hd = _make_pair_gather(N, D, PG)(h, sidx, didx)

    return _predictor(hs, hd, M, P, pp['W1'], pp['b1'], pp['W2'], pp['b2'],
                      pp['W3'], pp['b3'])
